# baseline (speedup 1.0000x reference)
import jax
import jax.numpy as jnp
from jax.experimental import pallas as pl
from jax.experimental.pallas import tpu as pltpu

_SLOPE = (1.0 / 8.0 + 1.0 / 3.0) / 2.0


def _fused_kernel(x_ref, wt_ref, wb_ref, b_ref, o_ref):
    x = x_ref[...]
    h1 = jnp.where(x >= 0, x, x * _SLOPE)
    h2 = jnp.where(x >= 0, x, x * (_SLOPE * _SLOPE))
    o_ref[...] = (
        jnp.dot(h1, wt_ref[...], preferred_element_type=jnp.float32)
        + jnp.dot(h2, wb_ref[...], preferred_element_type=jnp.float32)
        + b_ref[...]
    )


def kernel(node_feats, edge_feats, edge_index, Wn0, bn0, Wl0, bl0,
           Wn1, bn1, Wl1, bl1, Wo, bo):
    n, d = node_feats.shape
    h = Wo.shape[1]
    tile = 1000
    return pl.pallas_call(
        _fused_kernel,
        grid=(n // tile,),
        compiler_params=pltpu.CompilerParams(
            dimension_semantics=("parallel",),
        ),
        in_specs=[
            pl.BlockSpec((tile, d), lambda i: (i, 0)),
            pl.BlockSpec((d, h), lambda i: (0, 0)),
            pl.BlockSpec((d, h), lambda i: (0, 0)),
            pl.BlockSpec((1, h), lambda i: (0, 0)),
        ],
        out_specs=pl.BlockSpec((tile, h), lambda i: (i, 0)),
        out_shape=jax.ShapeDtypeStruct((n, h), jnp.float32),
    )(node_feats, Wo[:d], Wo[d:], bo.reshape(1, h))


# speedup vs baseline: 1.4910x; 1.4910x over previous
import jax
import jax.numpy as jnp
from jax.experimental import pallas as pl
from jax.experimental.pallas import tpu as pltpu

_SLOPE = (1.0 / 8.0 + 1.0 / 3.0) / 2.0


def _fused_kernel(x_ref, wt_ref, wb_ref, b_ref, o_ref):
    x = x_ref[...]
    h1 = jnp.where(x >= 0, x, x * _SLOPE)
    h2 = jnp.where(x >= 0, x, x * (_SLOPE * _SLOPE))
    o_ref[...] = (
        jnp.dot(h1, wt_ref[...], preferred_element_type=jnp.float32)
        + jnp.dot(h2, wb_ref[...], preferred_element_type=jnp.float32)
        + b_ref[...]
    )


def kernel(node_feats, edge_feats, edge_index, Wn0, bn0, Wl0, bl0,
           Wn1, bn1, Wl1, bl1, Wo, bo):
    n, d = node_feats.shape
    h = Wo.shape[1]
    tile = n
    return pl.pallas_call(
        _fused_kernel,
        grid=(n // tile,),
        compiler_params=pltpu.CompilerParams(
            dimension_semantics=("arbitrary",),
        ),
        in_specs=[
            pl.BlockSpec((tile, d), lambda i: (i, 0)),
            pl.BlockSpec((d, h), lambda i: (0, 0)),
            pl.BlockSpec((d, h), lambda i: (0, 0)),
            pl.BlockSpec((1, h), lambda i: (0, 0)),
        ],
        out_specs=pl.BlockSpec((tile, h), lambda i: (i, 0)),
        out_shape=jax.ShapeDtypeStruct((n, h), jnp.float32),
    )(node_feats, Wo[:d], Wo[d:], bo.reshape(1, h))
